# baseline (speedup 1.0000x reference)
import functools

import jax
import jax.numpy as jnp
from jax import lax
from jax.experimental import pallas as pl
from jax.experimental.pallas import tpu as pltpu
from jax.experimental.pallas import tpu_sc as plsc

_CUT1 = 20000
_CUT2 = 100000
_CUT3 = 1000000
_D = 128

_NC = 2
_NS = 16
_NW = _NC * _NS
_CHUNK = 128


def _sc_gather(idx, emb0, emb1, emb2):
    T = idx.shape[0]
    per_w = T // _NW
    n_chunks = per_w // _CHUNK
    mesh = plsc.VectorSubcoreMesh(core_axis_name="c", subcore_axis_name="s")

    @functools.partial(
        pl.kernel,
        out_type=(
            jax.ShapeDtypeStruct((T, 128), jnp.float32),
            jax.ShapeDtypeStruct((T, 32), jnp.float32),
            jax.ShapeDtypeStruct((T, 8), jnp.float32),
        ),
        mesh=mesh,
        scratch_types=[
            pltpu.VMEM((_CHUNK,), jnp.int32),
            pltpu.VMEM((_CHUNK,), jnp.int32),
            pltpu.VMEM((_CHUNK,), jnp.int32),
            pltpu.VMEM((_CHUNK,), jnp.int32),
            pltpu.VMEM((_CHUNK, 128), jnp.float32),
            pltpu.VMEM((_CHUNK, 32), jnp.float32),
            pltpu.VMEM((_CHUNK, 8), jnp.float32),
            pltpu.SemaphoreType.DMA,
            pltpu.SemaphoreType.DMA,
            pltpu.SemaphoreType.DMA,
        ],
        compiler_params=pltpu.CompilerParams(use_tc_tiling_on_sc=False),
    )
    def k(idx_hbm, e0, e1, e2, g0_hbm, g1_hbm, g2_hbm,
          idxv, i0, i1, i2, r0, r1, r2, s0, s1, s2):
        wid = lax.axis_index("s") * _NC + lax.axis_index("c")
        base = wid * per_w

        def chunk_body(ci, carry):
            off = base + ci * _CHUNK
            pltpu.sync_copy(idx_hbm.at[pl.ds(off, _CHUNK)], idxv)
            for vi in range(_CHUNK // 16):
                sl = pl.ds(vi * 16, 16)
                v = idxv[sl]
                i0[sl] = jnp.minimum(v, _CUT1 - 1)
                i1[sl] = jnp.clip(v - _CUT1, 0, _CUT2 - _CUT1 - 1)
                i2[sl] = jnp.maximum(v - _CUT2, 0)
            c0 = pltpu.async_copy(e0.at[i0], r0, s0)
            c1 = pltpu.async_copy(e1.at[i1], r1, s1)
            c2 = pltpu.async_copy(e2.at[i2], r2, s2)
            c0.wait()
            c1.wait()
            c2.wait()
            pltpu.sync_copy(r0, g0_hbm.at[pl.ds(off, _CHUNK)])
            pltpu.sync_copy(r1, g1_hbm.at[pl.ds(off, _CHUNK)])
            pltpu.sync_copy(r2, g2_hbm.at[pl.ds(off, _CHUNK)])
            return carry

        lax.fori_loop(0, n_chunks, chunk_body, 0)

    return k(idx, emb0, emb1, emb2)


def _tc_combine(idx2, g0, g1, g2, p0, p1, p2, bm):
    T = idx2.shape[0]

    def body(idx_ref, g0_ref, g1_ref, g2_ref, p0_ref, p1_ref, p2_ref, o_ref):
        iv = idx_ref[...]
        a = jnp.dot(g0_ref[...], p0_ref[...], preferred_element_type=jnp.float32)
        b = jnp.dot(g1_ref[...], p1_ref[...], preferred_element_type=jnp.float32)
        c = jnp.dot(g2_ref[...], p2_ref[...], preferred_element_type=jnp.float32)
        m0 = (iv < _CUT1).astype(jnp.float32)
        m01 = (iv < _CUT2).astype(jnp.float32)
        m1 = m01 - m0
        m2 = 1.0 - m01
        o_ref[...] = a * m0 + b * m1 + c * m2

    return pl.pallas_call(
        body,
        grid=(T // bm,),
        in_specs=[
            pl.BlockSpec((bm, 1), lambda i: (i, 0)),
            pl.BlockSpec((bm, 128), lambda i: (i, 0)),
            pl.BlockSpec((bm, 32), lambda i: (i, 0)),
            pl.BlockSpec((bm, 8), lambda i: (i, 0)),
            pl.BlockSpec((128, 128), lambda i: (0, 0)),
            pl.BlockSpec((32, 128), lambda i: (0, 0)),
            pl.BlockSpec((8, 128), lambda i: (0, 0)),
        ],
        out_specs=pl.BlockSpec((bm, _D), lambda i: (i, 0)),
        out_shape=jax.ShapeDtypeStruct((T, _D), jnp.float32),
    )(idx2, g0, g1, g2, p0, p1, p2)


def kernel(inputs, emb0, emb1, emb2, proj0, proj1, proj2):
    B, S = inputs.shape
    T = B * S
    idx = inputs.reshape(T).astype(jnp.int32)
    g0, g1, g2 = _sc_gather(idx, emb0, emb1, emb2)
    out = _tc_combine(idx.reshape(T, 1), g0, g1, g2, proj0, proj1, proj2,
                      bm=2048)
    return out.reshape(B, S, _D)


# speedup vs baseline: 7.9324x; 7.9324x over previous
import functools

import jax
import jax.numpy as jnp
from jax import lax
from jax.experimental import pallas as pl
from jax.experimental.pallas import tpu as pltpu
from jax.experimental.pallas import tpu_sc as plsc

_CUT1 = 20000
_CUT2 = 100000
_D = 128

_NC = 2
_NS = 16
_NW = _NC * _NS
_SEG = 256


def _sc_gather_padded(idx, emb0, emb1, emb2):
    T = idx.shape[0]
    per_w = T // _NW
    n_segs = per_w // _SEG
    mesh = plsc.VectorSubcoreMesh(core_axis_name="c", subcore_axis_name="s")

    @functools.partial(
        pl.kernel,
        out_type=jax.ShapeDtypeStruct((T * 128,), jnp.float32),
        mesh=mesh,
        scratch_types=[
            pltpu.VMEM((_SEG,), jnp.int32),
            pltpu.VMEM((_SEG + 16,), jnp.int32),
            pltpu.VMEM((_SEG + 16,), jnp.int32),
            pltpu.VMEM((_SEG + 16,), jnp.int32),
            pltpu.VMEM((_SEG + 16,), jnp.int32),
            pltpu.VMEM((_SEG,), jnp.int32),
            pltpu.VMEM((_SEG, 128), jnp.float32),
            pltpu.VMEM((_SEG, 32), jnp.float32),
            pltpu.VMEM((_SEG, 8), jnp.float32),
            pltpu.VMEM((_SEG * 128,), jnp.float32),
            pltpu.SemaphoreType.DMA,
            pltpu.SemaphoreType.DMA,
            pltpu.SemaphoreType.DMA,
        ],
        compiler_params=pltpu.CompilerParams(use_tc_tiling_on_sc=False, needs_layout_passes=False),
    )
    def k(idx_hbm, e0, e1, e2, g_hbm,
          idxv, i0, i1, p0, p1, i2, r0, r1, r2, gbuf, s0, s1, s2):
        wid = lax.axis_index("s") * _NC + lax.axis_index("c")
        base = wid * per_w
        zeros16 = jnp.zeros((16,), jnp.int32)
        iota16 = lax.broadcasted_iota(jnp.int32, (16,), 0)
        colv = iota16 & 7
        rowv = iota16 >> 3
        pat2 = colv + rowv * 128

        def seg_body(ci, carry):
            off = base + ci * _SEG
            pltpu.sync_copy(idx_hbm.at[pl.ds(off, _SEG)], idxv)
            for vi in range(_SEG // 16):
                sl = pl.ds(vi * 16, 16)
                i0[sl] = zeros16
                i1[sl] = zeros16
            n0 = jnp.int32(0)
            n1 = jnp.int32(0)
            for vi in range(_SEG // 16):
                sl = pl.ds(vi * 16, 16)
                v = idxv[sl]
                i2[sl] = jnp.maximum(v - _CUT2, 0)
                pos = iota16 + (vi * 16)
                m0v = v < _CUT1
                m1v = plsc.bitcast(v - _CUT1, jnp.uint32) < jnp.uint32(_CUT2 - _CUT1)
                c0 = plsc.all_reduce_population_count(m0v)[0]
                c1 = plsc.all_reduce_population_count(m1v)[0]
                plsc.store_compressed(i0.at[pl.ds(n0, 16)], jnp.minimum(v, _CUT1 - 1), mask=m0v)
                plsc.store_compressed(p0.at[pl.ds(n0, 16)], pos, mask=m0v)
                plsc.store_compressed(i1.at[pl.ds(n1, 16)], jnp.maximum(v - _CUT1, 0), mask=m1v)
                plsc.store_compressed(p1.at[pl.ds(n1, 16)], pos, mask=m1v)
                n0 = n0 + c0
                n1 = n1 + c1
            cp2 = pltpu.async_copy(e2.at[i2], r2, s2)

            def g0_body(gi, carry):
                pltpu.async_copy(
                    e0.at[i0.at[pl.ds(gi * 16, 16)]],
                    r0.at[pl.ds(gi * 16, 16)], s0).wait()
                return carry

            def g1_body(gi, carry):
                pltpu.async_copy(
                    e1.at[i1.at[pl.ds(gi * 16, 16)]],
                    r1.at[pl.ds(gi * 16, 16)], s1).wait()
                return carry

            lax.fori_loop(0, (n0 + 15) // 16, g0_body, 0)
            lax.fori_loop(0, (n1 + 15) // 16, g1_body, 0)
            cp2.wait()
            for j in range(_SEG // 2):
                vals = plsc.load_gather(r2, [rowv + (2 * j), colv])
                plsc.store_scatter(gbuf, [pat2 + (j * 256)], vals)

            def a0_body(j, carry):
                dst = p0[pl.ds(j, 16)][0] * 128
                for kk in range(8):
                    gbuf[pl.ds(dst + kk * 16, 16)] = r0[j, pl.ds(kk * 16, 16)]
                return carry

            def a1_body(j, carry):
                dst = p1[pl.ds(j, 16)][0] * 128
                for kk in range(2):
                    gbuf[pl.ds(dst + kk * 16, 16)] = r1[j, pl.ds(kk * 16, 16)]
                return carry

            lax.fori_loop(0, n0, a0_body, 0)
            lax.fori_loop(0, n1, a1_body, 0)
            pltpu.sync_copy(gbuf, g_hbm.at[pl.ds(off * 128, _SEG * 128)])
            return carry

        lax.fori_loop(0, n_segs, seg_body, 0)

    return k(idx, emb0, emb1, emb2)


def _tc_combine(idx2, g, p0z, p1z, p2z, bm):
    T = idx2.shape[0]

    def body(idx_ref, g_ref, p0_ref, p1_ref, p2_ref, o_ref):
        iv = idx_ref[...]
        gt = g_ref[...]
        a = jnp.dot(gt, p0_ref[...], preferred_element_type=jnp.float32)
        b = jnp.dot(gt, p1_ref[...], preferred_element_type=jnp.float32)
        c = jnp.dot(gt, p2_ref[...], preferred_element_type=jnp.float32)
        m0 = (iv < _CUT1).astype(jnp.float32)
        m01 = (iv < _CUT2).astype(jnp.float32)
        m1 = m01 - m0
        m2 = 1.0 - m01
        o_ref[...] = a * m0 + b * m1 + c * m2

    return pl.pallas_call(
        body,
        grid=(T // bm,),
        in_specs=[
            pl.BlockSpec((bm, 1), lambda i: (i, 0)),
            pl.BlockSpec((bm, 128), lambda i: (i, 0)),
            pl.BlockSpec((128, 128), lambda i: (0, 0)),
            pl.BlockSpec((128, 128), lambda i: (0, 0)),
            pl.BlockSpec((128, 128), lambda i: (0, 0)),
        ],
        out_specs=pl.BlockSpec((bm, _D), lambda i: (i, 0)),
        out_shape=jax.ShapeDtypeStruct((T, _D), jnp.float32),
    )(idx2, g, p0z, p1z, p2z)


def kernel(inputs, emb0, emb1, emb2, proj0, proj1, proj2):
    B, S = inputs.shape
    T = B * S
    idx = inputs.reshape(T).astype(jnp.int32)
    g = _sc_gather_padded(idx, emb0, emb1, emb2).reshape(T, 128)
    p1z = jnp.zeros((128, 128), jnp.float32).at[:32].set(proj1)
    p2z = jnp.zeros((128, 128), jnp.float32).at[:8].set(proj2)
    out = _tc_combine(idx.reshape(T, 1), g, proj0, p1z, p2z, bm=2048)
    return out.reshape(B, S, _D)


# speedup vs baseline: 7.9570x; 1.0031x over previous
import functools

import jax
import jax.numpy as jnp
from jax import lax
from jax.experimental import pallas as pl
from jax.experimental.pallas import tpu as pltpu
from jax.experimental.pallas import tpu_sc as plsc

_CUT1 = 20000
_CUT2 = 100000
_D = 128

_NC = 2
_NS = 16
_NW = _NC * _NS
_SEG = 256


def _sc_gather_padded(idx, emb0, emb1, emb2):
    T = idx.shape[0]
    per_w = T // _NW
    n_segs = per_w // _SEG
    mesh = plsc.VectorSubcoreMesh(core_axis_name="c", subcore_axis_name="s")

    @functools.partial(
        pl.kernel,
        out_type=jax.ShapeDtypeStruct((T * 128,), jnp.float32),
        mesh=mesh,
        scratch_types=[
            pltpu.VMEM((_SEG,), jnp.int32),
            pltpu.VMEM((_SEG + 16,), jnp.int32),
            pltpu.VMEM((_SEG + 16,), jnp.int32),
            pltpu.VMEM((_SEG + 16,), jnp.int32),
            pltpu.VMEM((_SEG + 16,), jnp.int32),
            pltpu.VMEM((_SEG,), jnp.int32),
            pltpu.VMEM((_SEG, 128), jnp.float32),
            pltpu.VMEM((_SEG, 32), jnp.float32),
            pltpu.VMEM((_SEG, 8), jnp.float32),
            pltpu.VMEM((_SEG * 128,), jnp.float32),
            pltpu.SemaphoreType.DMA,
            pltpu.SemaphoreType.DMA,
            pltpu.SemaphoreType.DMA,
        ],
        compiler_params=pltpu.CompilerParams(use_tc_tiling_on_sc=False, needs_layout_passes=False),
    )
    def k(idx_hbm, e0, e1, e2, g_hbm,
          idxv, i0, i1, p0, p1, i2, r0, r1, r2, gbuf, s0, s1, s2):
        wid = lax.axis_index("s") * _NC + lax.axis_index("c")
        base = wid * per_w
        zeros16 = jnp.zeros((16,), jnp.int32)
        iota16 = lax.broadcasted_iota(jnp.int32, (16,), 0)
        colv = iota16 & 7
        rowv = iota16 >> 3
        pat2 = colv + rowv * 128

        def seg_body(ci, carry):
            off = base + ci * _SEG
            pltpu.sync_copy(idx_hbm.at[pl.ds(off, _SEG)], idxv)
            for vi in range(_SEG // 16):
                sl = pl.ds(vi * 16, 16)
                i0[sl] = zeros16
                i1[sl] = zeros16
            n0 = jnp.int32(0)
            n1 = jnp.int32(0)
            for vi in range(_SEG // 16):
                sl = pl.ds(vi * 16, 16)
                v = idxv[sl]
                i2[sl] = jnp.maximum(v - _CUT2, 0)
                pos = iota16 + (vi * 16)
                m0v = v < _CUT1
                m1v = plsc.bitcast(v - _CUT1, jnp.uint32) < jnp.uint32(_CUT2 - _CUT1)
                c0 = plsc.all_reduce_population_count(m0v)[0]
                c1 = plsc.all_reduce_population_count(m1v)[0]
                plsc.store_compressed(i0.at[pl.ds(n0, 16)], jnp.minimum(v, _CUT1 - 1), mask=m0v)
                plsc.store_compressed(p0.at[pl.ds(n0, 16)], pos, mask=m0v)
                plsc.store_compressed(i1.at[pl.ds(n1, 16)], jnp.maximum(v - _CUT1, 0), mask=m1v)
                plsc.store_compressed(p1.at[pl.ds(n1, 16)], pos, mask=m1v)
                n0 = n0 + c0
                n1 = n1 + c1
            cp2 = pltpu.async_copy(e2.at[i2], r2, s2)

            def g0_body(gi, carry):
                pltpu.async_copy(
                    e0.at[i0.at[pl.ds(gi * 16, 16)]],
                    r0.at[pl.ds(gi * 16, 16)], s0).wait()
                return carry

            def g1_body(gi, carry):
                pltpu.async_copy(
                    e1.at[i1.at[pl.ds(gi * 16, 16)]],
                    r1.at[pl.ds(gi * 16, 16)], s1).wait()
                return carry

            lax.fori_loop(0, (n0 + 15) // 16, g0_body, 0)
            lax.fori_loop(0, (n1 + 15) // 16, g1_body, 0)
            cp2.wait()
            for j in range(_SEG // 2):
                vals = plsc.load_gather(r2, [rowv + (2 * j), colv])
                plsc.store_scatter(gbuf, [pat2 + (j * 256)], vals)

            def a0_body(j, carry):
                dst = p0[pl.ds(j, 16)][0] * 128
                for kk in range(8):
                    gbuf[pl.ds(dst + kk * 16, 16)] = r0[j, pl.ds(kk * 16, 16)]
                return carry

            def a1_body(j, carry):
                dst = p1[pl.ds(j, 16)][0] * 128
                for kk in range(2):
                    gbuf[pl.ds(dst + kk * 16, 16)] = r1[j, pl.ds(kk * 16, 16)]
                return carry

            lax.fori_loop(0, n0, a0_body, 0)
            lax.fori_loop(0, n1, a1_body, 0)
            pltpu.sync_copy(gbuf, g_hbm.at[pl.ds(off * 128, _SEG * 128)])
            return carry

        lax.fori_loop(0, n_segs, seg_body, 0)

    return k(idx, emb0, emb1, emb2)


def _tc_combine(idx2, g, p0z, p1z, p2z, bm):
    T = idx2.shape[0]

    def body(idx_ref, g_ref, p0_ref, p1_ref, p2_ref, o_ref):
        iv = idx_ref[...]
        gt = g_ref[...].astype(jnp.bfloat16)
        a = jnp.dot(gt, p0_ref[...].astype(jnp.bfloat16),
                    preferred_element_type=jnp.float32)
        b = jnp.dot(gt, p1_ref[...].astype(jnp.bfloat16),
                    preferred_element_type=jnp.float32)
        c = jnp.dot(gt, p2_ref[...].astype(jnp.bfloat16),
                    preferred_element_type=jnp.float32)
        m0 = (iv < _CUT1).astype(jnp.float32)
        m01 = (iv < _CUT2).astype(jnp.float32)
        m1 = m01 - m0
        m2 = 1.0 - m01
        o_ref[...] = a * m0 + b * m1 + c * m2

    return pl.pallas_call(
        body,
        grid=(T // bm,),
        in_specs=[
            pl.BlockSpec((bm, 1), lambda i: (i, 0)),
            pl.BlockSpec((bm, 128), lambda i: (i, 0)),
            pl.BlockSpec((128, 128), lambda i: (0, 0)),
            pl.BlockSpec((128, 128), lambda i: (0, 0)),
            pl.BlockSpec((128, 128), lambda i: (0, 0)),
        ],
        out_specs=pl.BlockSpec((bm, _D), lambda i: (i, 0)),
        out_shape=jax.ShapeDtypeStruct((T, _D), jnp.float32),
    )(idx2, g, p0z, p1z, p2z)


def kernel(inputs, emb0, emb1, emb2, proj0, proj1, proj2):
    B, S = inputs.shape
    T = B * S
    idx = inputs.reshape(T).astype(jnp.int32)
    g = _sc_gather_padded(idx, emb0, emb1, emb2).reshape(T, 128)
    p1z = jnp.zeros((128, 128), jnp.float32).at[:32].set(proj1)
    p2z = jnp.zeros((128, 128), jnp.float32).at[:8].set(proj2)
    out = _tc_combine(idx.reshape(T, 1), g, proj0, p1z, p2z, bm=2048)
    return out.reshape(B, S, _D)


# speedup vs baseline: 7.9842x; 1.0034x over previous
import functools

import jax
import jax.numpy as jnp
from jax import lax
from jax.experimental import pallas as pl
from jax.experimental.pallas import tpu as pltpu
from jax.experimental.pallas import tpu_sc as plsc

_CUT1 = 20000
_CUT2 = 100000
_D = 128

_NC = 2
_NS = 16
_NW = _NC * _NS
_SEG = 256


def _sc_gather_padded(idx, emb0, emb1, emb2):
    T = idx.shape[0]
    per_w = T // _NW
    n_segs = per_w // _SEG
    mesh = plsc.VectorSubcoreMesh(core_axis_name="c", subcore_axis_name="s")

    @functools.partial(
        pl.kernel,
        out_type=jax.ShapeDtypeStruct((T, 128), jnp.float32),
        mesh=mesh,
        scratch_types=[
            pltpu.VMEM((_SEG,), jnp.int32),
            pltpu.VMEM((_SEG + 16,), jnp.int32),
            pltpu.VMEM((_SEG + 16,), jnp.int32),
            pltpu.VMEM((_SEG + 16,), jnp.int32),
            pltpu.VMEM((_SEG + 16,), jnp.int32),
            pltpu.VMEM((_SEG,), jnp.int32),
            pltpu.VMEM((_SEG, 128), jnp.float32),
            pltpu.VMEM((_SEG, 32), jnp.float32),
            pltpu.VMEM((_SEG, 8), jnp.float32),
            pltpu.VMEM((_SEG, 128), jnp.float32),
            pltpu.SemaphoreType.DMA,
            pltpu.SemaphoreType.DMA,
            pltpu.SemaphoreType.DMA,
        ],
        compiler_params=pltpu.CompilerParams(use_tc_tiling_on_sc=False, needs_layout_passes=False),
    )
    def k(idx_hbm, e0, e1, e2, g_hbm,
          idxv, i0, i1, p0, p1, i2, r0, r1, r2, gbuf, s0, s1, s2):
        wid = lax.axis_index("s") * _NC + lax.axis_index("c")
        base = wid * per_w
        zeros16 = jnp.zeros((16,), jnp.int32)
        iota16 = lax.broadcasted_iota(jnp.int32, (16,), 0)
        colv = iota16 & 7
        rowv = iota16 >> 3

        def seg_body(ci, carry):
            off = base + ci * _SEG
            pltpu.sync_copy(idx_hbm.at[pl.ds(off, _SEG)], idxv)
            for vi in range(_SEG // 16):
                sl = pl.ds(vi * 16, 16)
                i0[sl] = zeros16
                i1[sl] = zeros16
            n0 = jnp.int32(0)
            n1 = jnp.int32(0)
            for vi in range(_SEG // 16):
                sl = pl.ds(vi * 16, 16)
                v = idxv[sl]
                i2[sl] = jnp.maximum(v - _CUT2, 0)
                pos = iota16 + (vi * 16)
                m0v = v < _CUT1
                m1v = plsc.bitcast(v - _CUT1, jnp.uint32) < jnp.uint32(_CUT2 - _CUT1)
                c0 = plsc.all_reduce_population_count(m0v)[0]
                c1 = plsc.all_reduce_population_count(m1v)[0]
                plsc.store_compressed(i0.at[pl.ds(n0, 16)], jnp.minimum(v, _CUT1 - 1), mask=m0v)
                plsc.store_compressed(p0.at[pl.ds(n0, 16)], pos, mask=m0v)
                plsc.store_compressed(i1.at[pl.ds(n1, 16)], jnp.maximum(v - _CUT1, 0), mask=m1v)
                plsc.store_compressed(p1.at[pl.ds(n1, 16)], pos, mask=m1v)
                n0 = n0 + c0
                n1 = n1 + c1
            cp2 = pltpu.async_copy(e2.at[i2], r2, s2)

            def g0_body(gi, carry):
                pltpu.async_copy(
                    e0.at[i0.at[pl.ds(gi * 16, 16)]],
                    r0.at[pl.ds(gi * 16, 16)], s0).wait()
                return carry

            def g1_body(gi, carry):
                pltpu.async_copy(
                    e1.at[i1.at[pl.ds(gi * 16, 16)]],
                    r1.at[pl.ds(gi * 16, 16)], s1).wait()
                return carry

            lax.fori_loop(0, (n0 + 15) // 16, g0_body, 0)
            lax.fori_loop(0, (n1 + 15) // 16, g1_body, 0)
            cp2.wait()
            for j in range(_SEG // 2):
                vals = plsc.load_gather(r2, [rowv + (2 * j), colv])
                plsc.store_scatter(gbuf, [rowv + (2 * j), colv], vals)

            def a0_body(j, carry):
                dst = p0[pl.ds(j, 16)][0]
                for kk in range(8):
                    gbuf[dst, pl.ds(kk * 16, 16)] = r0[j, pl.ds(kk * 16, 16)]
                return carry

            def a1_body(j, carry):
                dst = p1[pl.ds(j, 16)][0]
                for kk in range(2):
                    gbuf[dst, pl.ds(kk * 16, 16)] = r1[j, pl.ds(kk * 16, 16)]
                return carry

            lax.fori_loop(0, n0, a0_body, 0)
            lax.fori_loop(0, n1, a1_body, 0)
            pltpu.sync_copy(gbuf, g_hbm.at[pl.ds(off, _SEG)])
            return carry

        lax.fori_loop(0, n_segs, seg_body, 0)

    return k(idx, emb0, emb1, emb2)


def _tc_combine(idx2, g, p0z, p1z, p2z, bm):
    T = idx2.shape[0]

    def body(idx_ref, g_ref, p0_ref, p1_ref, p2_ref, o_ref):
        iv = idx_ref[...]
        gt = g_ref[...].astype(jnp.bfloat16)
        a = jnp.dot(gt, p0_ref[...].astype(jnp.bfloat16),
                    preferred_element_type=jnp.float32)
        b = jnp.dot(gt, p1_ref[...].astype(jnp.bfloat16),
                    preferred_element_type=jnp.float32)
        c = jnp.dot(gt, p2_ref[...].astype(jnp.bfloat16),
                    preferred_element_type=jnp.float32)
        m0 = (iv < _CUT1).astype(jnp.float32)
        m01 = (iv < _CUT2).astype(jnp.float32)
        m1 = m01 - m0
        m2 = 1.0 - m01
        o_ref[...] = a * m0 + b * m1 + c * m2

    return pl.pallas_call(
        body,
        grid=(T // bm,),
        in_specs=[
            pl.BlockSpec((bm, 1), lambda i: (i, 0)),
            pl.BlockSpec((bm, 128), lambda i: (i, 0)),
            pl.BlockSpec((128, 128), lambda i: (0, 0)),
            pl.BlockSpec((128, 128), lambda i: (0, 0)),
            pl.BlockSpec((128, 128), lambda i: (0, 0)),
        ],
        out_specs=pl.BlockSpec((bm, _D), lambda i: (i, 0)),
        out_shape=jax.ShapeDtypeStruct((T, _D), jnp.float32),
    )(idx2, g, p0z, p1z, p2z)


def kernel(inputs, emb0, emb1, emb2, proj0, proj1, proj2):
    B, S = inputs.shape
    T = B * S
    idx = inputs.reshape(T).astype(jnp.int32)
    g = _sc_gather_padded(idx, emb0, emb1, emb2)
    p1z = jnp.zeros((128, 128), jnp.float32).at[:32].set(proj1)
    p2z = jnp.zeros((128, 128), jnp.float32).at[:8].set(proj2)
    out = _tc_combine(idx.reshape(T, 1), g, proj0, p1z, p2z, bm=2048)
    return out.reshape(B, S, _D)
